# baseline (speedup 1.0000x reference)
import math

import jax
import jax.numpy as jnp
from jax.experimental import pallas as pl
from jax.experimental.pallas import tpu as pltpu

_GRID = 192
_SCALE = 0.25
_IMG = 768.0
_OUT = 7
_SR = 2
_NS = _OUT * _SR
_K = 64
_POS_D = 720

_BC0, _BC1, _BC2, _BC3 = 16, 64, 256, 256
_N0, _N1, _N2, _N3 = 192 // _BC0, 384 // _BC1, 768 // _BC2, 1536 // _BC3
_S1 = _N0
_S2 = _S1 + _N1
_S3 = _S2 + _N2
_STEPS = _S3 + _N3


def _up_tent(n, rows, row_iota_dim, col_iota_dim, wdiv=None):
    i = jax.lax.broadcasted_iota(jnp.int32, (rows, _GRID),
                                 col_iota_dim).astype(jnp.float32)
    u = jnp.clip((i + 0.5) * (n / float(_GRID)) - 0.5, 0.0, float(n - 1))
    p = jax.lax.broadcasted_iota(jnp.int32, (rows, _GRID), row_iota_dim)
    if wdiv is not None:
        p = p // wdiv
    r = p.astype(jnp.float32)
    return jnp.maximum(0.0, 1.0 - jnp.abs(u - r))


def _up_tent_mod(n, w, rows):
    i = jax.lax.broadcasted_iota(jnp.int32, (rows, _GRID),
                                 1).astype(jnp.float32)
    u = jnp.clip((i + 0.5) * (n / float(_GRID)) - 0.5, 0.0, float(n - 1))
    p = jax.lax.broadcasted_iota(jnp.int32, (rows, _GRID), 0)
    q = (p - (p // w) * w).astype(jnp.float32)
    return jnp.maximum(0.0, 1.0 - jnp.abs(u - q))


def _body(b_ref, f0_ref, f1_ref, f2_ref, f3_ref, o_ref,
          bt_ref, acc_ref, ay_ref, ax_ref, rw0_ref, cw0_ref, rw1_ref,
          cw1_ref, w2_ref, w3_ref):
    i = pl.program_id(0)

    @pl.when(i == 0)
    def _init_bt():
        bt_ref[...] = jnp.transpose(b_ref[...], (1, 0))

    bT = bt_ref[...]

    @pl.when(i == 0)
    def _init():
        c = jax.lax.broadcasted_iota(jnp.int32, (2880, 1), 0)
        blk = c // _POS_D
        j = c - blk * _POS_D
        expo = (2.0 * (j // 2).astype(jnp.float32)) / float(_POS_D)
        inv_t = jnp.exp(-math.log(10000.0) * expo)
        nx1 = bT[0:1, :] / _IMG
        ny1 = bT[1:2, :] / _IMG
        nw = bT[2:3, :] / _IMG - nx1
        nh = bT[3:4, :] / _IMG - ny1
        v = jnp.where(blk == 0, ny1 + nh * 0.5,
                      jnp.where(blk == 1, nx1 + nw * 0.5,
                                jnp.where(blk == 2, nw, nh)))
        ang = v * (2.0 * math.pi) * inv_t
        acc_ref[...] = jnp.where(j % 2 == 0, jnp.sin(ang), jnp.cos(ang))
        x1 = bT[0:1, :] * _SCALE
        y1 = bT[1:2, :] * _SCALE
        x2 = bT[2:3, :] * _SCALE
        y2 = bT[3:4, :] * _SCALE
        roi_w = jnp.maximum(x2 - x1, 1.0)
        roi_h = jnp.maximum(y2 - y1, 1.0)
        g = (jax.lax.broadcasted_iota(jnp.int32, (_NS, _K), 0)
             .astype(jnp.float32) + 0.5) / (2.0 * _OUT)
        xs = x1 + g * roi_w
        ys = y1 + g * roi_h

        def grid_tent(s):
            valid = ((s > -1.0) & (s < float(_GRID))).astype(jnp.float32)
            sc = jnp.clip(s, 0.0, float(_GRID - 1))
            gi = jax.lax.broadcasted_iota(jnp.int32, (_GRID, _NS, _K),
                                          0).astype(jnp.float32)
            t = jnp.maximum(0.0, 1.0 - jnp.abs(sc[None] - gi)) * valid[None]
            return jnp.sum(t, axis=1) * (1.0 / _NS)

        ay_ref[...] = grid_tent(ys)
        ax_ref[...] = grid_tent(xs)
        t0 = _up_tent(192, 192, 0, 1)
        rw0_ref[...] = jnp.dot(t0, ay_ref[...],
                               preferred_element_type=jnp.float32)
        cw0_ref[...] = jnp.dot(t0, ax_ref[...],
                               preferred_element_type=jnp.float32)

    @pl.when(i < _S1)
    def _level0():
        x = f0_ref[...].reshape(_BC0 * 192, 192)
        b = jnp.dot(x, cw0_ref[...], preferred_element_type=jnp.float32)
        pooled = jnp.sum(b.reshape(_BC0, 192, _K) * rw0_ref[...][None],
                         axis=1)
        base = i * _BC0
        acc_ref[pl.ds(base, _BC0), :] += pooled

    @pl.when((i >= _S1) & (i < _S2))
    def _level1():
        @pl.when(i == _S1)
        def _():
            t1 = _up_tent(96, 96, 0, 1)
            rw1_ref[...] = jnp.dot(t1, ay_ref[...],
                                   preferred_element_type=jnp.float32)
            cw1_ref[...] = jnp.dot(t1, ax_ref[...],
                                   preferred_element_type=jnp.float32)
        x = f1_ref[...].reshape(_BC1 * 96, 96)
        b = jnp.dot(x, cw1_ref[...], preferred_element_type=jnp.float32)
        pooled = jnp.sum(b.reshape(_BC1, 96, _K) * rw1_ref[...][None],
                         axis=1)
        base = 192 + (i - _S1) * _BC1
        acc_ref[pl.ds(base, _BC1), :] += pooled

    @pl.when((i >= _S2) & (i < _S3))
    def _level2():
        @pl.when(i == _S2)
        def _():
            ty = _up_tent(48, 2304, 0, 1, wdiv=48)
            tx = _up_tent_mod(48, 48, 2304)
            w2_ref[...] = (
                jnp.dot(ty, ay_ref[...], preferred_element_type=jnp.float32)
                * jnp.dot(tx, ax_ref[...],
                          preferred_element_type=jnp.float32))
        x = f2_ref[...].reshape(_BC2, 2304)
        pooled = jnp.dot(x, w2_ref[...], preferred_element_type=jnp.float32)
        base = 576 + (i - _S2) * _BC2
        acc_ref[pl.ds(base, _BC2), :] += pooled

    @pl.when(i >= _S3)
    def _level3():
        @pl.when(i == _S3)
        def _():
            ty = _up_tent(24, 576, 0, 1, wdiv=24)
            tx = _up_tent_mod(24, 24, 576)
            w3_ref[...] = (
                jnp.dot(ty, ay_ref[...], preferred_element_type=jnp.float32)
                * jnp.dot(tx, ax_ref[...],
                          preferred_element_type=jnp.float32))
        x = f3_ref[...].reshape(_BC3, 576)
        pooled = jnp.dot(x, w3_ref[...], preferred_element_type=jnp.float32)
        base = 1344 + (i - _S3) * _BC3
        acc_ref[pl.ds(base, _BC3), :] += pooled

    @pl.when(i == _STEPS - 1)
    def _emit():
        o_ref[...] = jnp.transpose(acc_ref[...], (1, 0))


def kernel(feat0, feat1, feat2, feat3, boxes):
    f2 = feat2.reshape(1, 768, 48 * 48)
    f3 = feat3.reshape(1, 1536, 24 * 24)
    full = pl.pallas_call(
        _body,
        out_shape=jax.ShapeDtypeStruct((_K, 2880), jnp.float32),
        grid=(_STEPS,),
        in_specs=[
            pl.BlockSpec((_K, 4), lambda i: (0, 0)),
            pl.BlockSpec((1, _BC0, 192, 192),
                         lambda i: (0, jnp.minimum(i, _S1 - 1), 0, 0)),
            pl.BlockSpec((1, _BC1, 96, 96),
                         lambda i: (0, jnp.clip(i - _S1, 0, _N1 - 1), 0, 0)),
            pl.BlockSpec((1, _BC2, 2304),
                         lambda i: (0, jnp.clip(i - _S2, 0, _N2 - 1), 0)),
            pl.BlockSpec((1, _BC3, 576),
                         lambda i: (0, jnp.clip(i - _S3, 0, _N3 - 1), 0)),
        ],
        out_specs=pl.BlockSpec((_K, 2880), lambda i: (0, 0)),
        scratch_shapes=[
            pltpu.VMEM((4, _K), jnp.float32),
            pltpu.VMEM((2880, _K), jnp.float32),
            pltpu.VMEM((192, _K), jnp.float32),
            pltpu.VMEM((192, _K), jnp.float32),
            pltpu.VMEM((192, _K), jnp.float32),
            pltpu.VMEM((192, _K), jnp.float32),
            pltpu.VMEM((96, _K), jnp.float32),
            pltpu.VMEM((96, _K), jnp.float32),
            pltpu.VMEM((2304, _K), jnp.float32),
            pltpu.VMEM((576, _K), jnp.float32),
        ],
        compiler_params=pltpu.CompilerParams(
            dimension_semantics=("arbitrary",)),
        name="roi_fused",
    )(boxes, feat0, feat1, f2, f3)
    return full[None]


# speedup vs baseline: 1.0807x; 1.0807x over previous
import math

import jax
import jax.numpy as jnp
from jax.experimental import pallas as pl
from jax.experimental.pallas import tpu as pltpu

_GRID = 192
_SCALE = 0.25
_IMG = 768.0
_OUT = 7
_SR = 2
_NS = _OUT * _SR
_K = 64
_POS_D = 720

_BC0, _BC1, _BC2, _BC3 = 16, 64, 256, 256
_N0, _N1, _N2, _N3 = 192 // _BC0, 384 // _BC1, 768 // _BC2, 1536 // _BC3
_S1 = _N0
_S2 = _S1 + _N1
_S3 = _S2 + _N2
_STEPS = _S3 + _N3


def _up_tent(n, rows, row_iota_dim, col_iota_dim, wdiv=None):
    i = jax.lax.broadcasted_iota(jnp.int32, (rows, _GRID),
                                 col_iota_dim).astype(jnp.float32)
    u = jnp.clip((i + 0.5) * (n / float(_GRID)) - 0.5, 0.0, float(n - 1))
    p = jax.lax.broadcasted_iota(jnp.int32, (rows, _GRID), row_iota_dim)
    if wdiv is not None:
        p = p // wdiv
    r = p.astype(jnp.float32)
    return jnp.maximum(0.0, 1.0 - jnp.abs(u - r))


def _up_tent_mod(n, w, rows):
    i = jax.lax.broadcasted_iota(jnp.int32, (rows, _GRID),
                                 1).astype(jnp.float32)
    u = jnp.clip((i + 0.5) * (n / float(_GRID)) - 0.5, 0.0, float(n - 1))
    p = jax.lax.broadcasted_iota(jnp.int32, (rows, _GRID), 0)
    q = (p - (p // w) * w).astype(jnp.float32)
    return jnp.maximum(0.0, 1.0 - jnp.abs(u - q))


def _body(b_ref, f0_ref, f1_ref, f2_ref, f3_ref, o_ref,
          bt_ref, acc_ref, ay_ref, ax_ref, rw0_ref, cw0_ref, rw1_ref,
          cw1_ref, w2_ref, w3_ref):
    i = pl.program_id(0)

    @pl.when(i == 0)
    def _init_bt():
        bt_ref[...] = jnp.transpose(b_ref[...], (1, 0))

    bT = bt_ref[...]

    def pos_embed(base, bc):
        c = base + jax.lax.broadcasted_iota(jnp.int32, (bc, 1), 0)
        blk = c // _POS_D
        j = c - blk * _POS_D
        expo = (2.0 * (j // 2).astype(jnp.float32)) / float(_POS_D)
        inv_t = jnp.exp(-math.log(10000.0) * expo)
        nx1 = bT[0:1, :] / _IMG
        ny1 = bT[1:2, :] / _IMG
        nw = bT[2:3, :] / _IMG - nx1
        nh = bT[3:4, :] / _IMG - ny1
        v = jnp.where(blk == 0, ny1 + nh * 0.5,
                      jnp.where(blk == 1, nx1 + nw * 0.5,
                                jnp.where(blk == 2, nw, nh)))
        ang = v * (2.0 * math.pi) * inv_t
        return jnp.where(j % 2 == 0, jnp.sin(ang), jnp.cos(ang))

    @pl.when(i == 0)
    def _init():
        x1 = bT[0:1, :] * _SCALE
        y1 = bT[1:2, :] * _SCALE
        x2 = bT[2:3, :] * _SCALE
        y2 = bT[3:4, :] * _SCALE
        roi_w = jnp.maximum(x2 - x1, 1.0)
        roi_h = jnp.maximum(y2 - y1, 1.0)
        g = (jax.lax.broadcasted_iota(jnp.int32, (_NS, _K), 0)
             .astype(jnp.float32) + 0.5) / (2.0 * _OUT)
        xs = x1 + g * roi_w
        ys = y1 + g * roi_h

        def grid_tent(s):
            valid = ((s > -1.0) & (s < float(_GRID))).astype(jnp.float32)
            sc = jnp.clip(s, 0.0, float(_GRID - 1))
            gi = jax.lax.broadcasted_iota(jnp.int32, (_GRID, _NS, _K),
                                          0).astype(jnp.float32)
            t = jnp.maximum(0.0, 1.0 - jnp.abs(sc[None] - gi)) * valid[None]
            return jnp.sum(t, axis=1) * (1.0 / _NS)

        ay_ref[...] = grid_tent(ys)
        ax_ref[...] = grid_tent(xs)
        t0 = _up_tent(192, 192, 0, 1)
        rw0_ref[...] = jnp.dot(t0, ay_ref[...],
                               preferred_element_type=jnp.float32)
        cw0_ref[...] = jnp.dot(t0, ax_ref[...],
                               preferred_element_type=jnp.float32)

    @pl.when(i < _S1)
    def _level0():
        x = f0_ref[...].reshape(_BC0 * 192, 192)
        b = jnp.dot(x, cw0_ref[...], preferred_element_type=jnp.float32)
        pooled = jnp.sum(b.reshape(_BC0, 192, _K) * rw0_ref[...][None],
                         axis=1)
        base = i * _BC0
        acc_ref[pl.ds(base, _BC0), :] = pooled + pos_embed(base, _BC0)

    @pl.when((i >= _S1) & (i < _S2))
    def _level1():
        @pl.when(i == _S1)
        def _():
            t1 = _up_tent(96, 96, 0, 1)
            rw1_ref[...] = jnp.dot(t1, ay_ref[...],
                                   preferred_element_type=jnp.float32)
            cw1_ref[...] = jnp.dot(t1, ax_ref[...],
                                   preferred_element_type=jnp.float32)
        x = f1_ref[...].reshape(_BC1 * 96, 96)
        b = jnp.dot(x, cw1_ref[...], preferred_element_type=jnp.float32)
        pooled = jnp.sum(b.reshape(_BC1, 96, _K) * rw1_ref[...][None],
                         axis=1)
        base = 192 + (i - _S1) * _BC1
        acc_ref[pl.ds(base, _BC1), :] = pooled + pos_embed(base, _BC1)

    @pl.when((i >= _S2) & (i < _S3))
    def _level2():
        @pl.when(i == _S2)
        def _():
            ty = _up_tent(48, 2304, 0, 1, wdiv=48)
            tx = _up_tent_mod(48, 48, 2304)
            w2_ref[...] = (
                jnp.dot(ty, ay_ref[...], preferred_element_type=jnp.float32)
                * jnp.dot(tx, ax_ref[...],
                          preferred_element_type=jnp.float32))
        x = f2_ref[...].reshape(_BC2, 2304)
        pooled = jnp.dot(x, w2_ref[...], preferred_element_type=jnp.float32)
        base = 576 + (i - _S2) * _BC2
        acc_ref[pl.ds(base, _BC2), :] = pooled + pos_embed(base, _BC2)

    @pl.when(i >= _S3)
    def _level3():
        @pl.when(i == _S3)
        def _():
            ty = _up_tent(24, 576, 0, 1, wdiv=24)
            tx = _up_tent_mod(24, 24, 576)
            w3_ref[...] = (
                jnp.dot(ty, ay_ref[...], preferred_element_type=jnp.float32)
                * jnp.dot(tx, ax_ref[...],
                          preferred_element_type=jnp.float32))
        x = f3_ref[...].reshape(_BC3, 576)
        pooled = jnp.dot(x, w3_ref[...], preferred_element_type=jnp.float32)
        base = 1344 + (i - _S3) * _BC3
        acc_ref[pl.ds(base, _BC3), :] = pooled + pos_embed(base, _BC3)

    @pl.when(i == _STEPS - 1)
    def _emit():
        o_ref[...] = jnp.transpose(acc_ref[...], (1, 0))


def kernel(feat0, feat1, feat2, feat3, boxes):
    f2 = feat2.reshape(1, 768, 48 * 48)
    f3 = feat3.reshape(1, 1536, 24 * 24)
    full = pl.pallas_call(
        _body,
        out_shape=jax.ShapeDtypeStruct((_K, 2880), jnp.float32),
        grid=(_STEPS,),
        in_specs=[
            pl.BlockSpec((_K, 4), lambda i: (0, 0)),
            pl.BlockSpec((1, _BC0, 192, 192),
                         lambda i: (0, jnp.minimum(i, _S1 - 1), 0, 0)),
            pl.BlockSpec((1, _BC1, 96, 96),
                         lambda i: (0, jnp.clip(i - _S1, 0, _N1 - 1), 0, 0)),
            pl.BlockSpec((1, _BC2, 2304),
                         lambda i: (0, jnp.clip(i - _S2, 0, _N2 - 1), 0)),
            pl.BlockSpec((1, _BC3, 576),
                         lambda i: (0, jnp.clip(i - _S3, 0, _N3 - 1), 0)),
        ],
        out_specs=pl.BlockSpec((_K, 2880), lambda i: (0, 0)),
        scratch_shapes=[
            pltpu.VMEM((4, _K), jnp.float32),
            pltpu.VMEM((2880, _K), jnp.float32),
            pltpu.VMEM((192, _K), jnp.float32),
            pltpu.VMEM((192, _K), jnp.float32),
            pltpu.VMEM((192, _K), jnp.float32),
            pltpu.VMEM((192, _K), jnp.float32),
            pltpu.VMEM((96, _K), jnp.float32),
            pltpu.VMEM((96, _K), jnp.float32),
            pltpu.VMEM((2304, _K), jnp.float32),
            pltpu.VMEM((576, _K), jnp.float32),
        ],
        compiler_params=pltpu.CompilerParams(
            dimension_semantics=("arbitrary",)),
        name="roi_fused",
    )(boxes, feat0, feat1, f2, f3)
    return full[None]


# speedup vs baseline: 1.0821x; 1.0013x over previous
import math

import jax
import jax.numpy as jnp
from jax.experimental import pallas as pl
from jax.experimental.pallas import tpu as pltpu

_GRID = 192
_SCALE = 0.25
_IMG = 768.0
_OUT = 7
_SR = 2
_NS = _OUT * _SR
_K = 64
_POS_D = 720

_BC0, _BC1, _BC2, _BC3 = 16, 64, 256, 256
_N0, _N1, _N2, _N3 = 192 // _BC0, 384 // _BC1, 768 // _BC2, 1536 // _BC3
_S1 = _N0
_S2 = _S1 + _N1
_S3 = _S2 + _N2
_STEPS = _S3 + _N3


def _up_tent(n, rows, row_iota_dim, col_iota_dim, wdiv=None):
    i = jax.lax.broadcasted_iota(jnp.int32, (rows, _GRID),
                                 col_iota_dim).astype(jnp.float32)
    u = jnp.clip((i + 0.5) * (n / float(_GRID)) - 0.5, 0.0, float(n - 1))
    p = jax.lax.broadcasted_iota(jnp.int32, (rows, _GRID), row_iota_dim)
    if wdiv is not None:
        p = p // wdiv
    r = p.astype(jnp.float32)
    return jnp.maximum(0.0, 1.0 - jnp.abs(u - r))


def _up_tent_mod(n, w, rows):
    i = jax.lax.broadcasted_iota(jnp.int32, (rows, _GRID),
                                 1).astype(jnp.float32)
    u = jnp.clip((i + 0.5) * (n / float(_GRID)) - 0.5, 0.0, float(n - 1))
    p = jax.lax.broadcasted_iota(jnp.int32, (rows, _GRID), 0)
    q = (p - (p // w) * w).astype(jnp.float32)
    return jnp.maximum(0.0, 1.0 - jnp.abs(u - q))


def _body(b_ref, f0_ref, f1_ref, f2_ref, f3_ref, o_ref,
          bt_ref, acc_ref, ay_ref, ax_ref, rw0_ref, cw0_ref, rw1_ref,
          cw1_ref, w2_ref, w3_ref):
    i = pl.program_id(0)

    @pl.when(i == 0)
    def _init_bt():
        bt_ref[...] = jnp.transpose(b_ref[...], (1, 0))

    bT = bt_ref[...]

    def pos_embed(base, bc):
        c = base + jax.lax.broadcasted_iota(jnp.int32, (bc, 1), 0)
        blk = c // _POS_D
        j = c - blk * _POS_D
        expo = (2.0 * (j // 2).astype(jnp.float32)) / float(_POS_D)
        inv_t = jnp.exp(-math.log(10000.0) * expo)
        nx1 = bT[0:1, :] / _IMG
        ny1 = bT[1:2, :] / _IMG
        nw = bT[2:3, :] / _IMG - nx1
        nh = bT[3:4, :] / _IMG - ny1
        v = jnp.where(blk == 0, ny1 + nh * 0.5,
                      jnp.where(blk == 1, nx1 + nw * 0.5,
                                jnp.where(blk == 2, nw, nh)))
        ang = v * (2.0 * math.pi) * inv_t
        return jnp.where(j % 2 == 0, jnp.sin(ang), jnp.cos(ang))

    @pl.when(i == 0)
    def _init():
        x1 = bT[0:1, :] * _SCALE
        y1 = bT[1:2, :] * _SCALE
        x2 = bT[2:3, :] * _SCALE
        y2 = bT[3:4, :] * _SCALE
        roi_w = jnp.maximum(x2 - x1, 1.0)
        roi_h = jnp.maximum(y2 - y1, 1.0)
        g = (jax.lax.broadcasted_iota(jnp.int32, (_NS, _K), 0)
             .astype(jnp.float32) + 0.5) / (2.0 * _OUT)
        xs = x1 + g * roi_w
        ys = y1 + g * roi_h

        def grid_tent(s):
            valid = ((s > -1.0) & (s < float(_GRID))).astype(jnp.float32)
            sc = jnp.clip(s, 0.0, float(_GRID - 1))
            gi = jax.lax.broadcasted_iota(jnp.int32, (_GRID, _NS, _K),
                                          0).astype(jnp.float32)
            t = jnp.maximum(0.0, 1.0 - jnp.abs(sc[None] - gi)) * valid[None]
            return jnp.sum(t, axis=1) * (1.0 / _NS)

        ay_ref[...] = grid_tent(ys)
        ax_ref[...] = grid_tent(xs)
        t0 = _up_tent(192, 192, 0, 1)
        rw0_ref[...] = jnp.dot(t0, ay_ref[...],
                               preferred_element_type=jnp.float32)
        cw0_ref[...] = jnp.dot(t0, ax_ref[...],
                               preferred_element_type=jnp.float32)

    @pl.when(i < _S1)
    def _level0():
        base = i * _BC0
        for s in range(0, _BC0, 8):
            x = f0_ref[0, s:s + 8].reshape(8 * 192, 192)
            b = jnp.dot(x, cw0_ref[...], preferred_element_type=jnp.float32)
            pooled = jnp.sum(b.reshape(8, 192, _K) * rw0_ref[...][None],
                             axis=1)
            acc_ref[pl.ds(base + s, 8), :] = pooled + pos_embed(base + s, 8)

    @pl.when((i >= _S1) & (i < _S2))
    def _level1():
        @pl.when(i == _S1)
        def _():
            t1 = _up_tent(96, 96, 0, 1)
            rw1_ref[...] = jnp.dot(t1, ay_ref[...],
                                   preferred_element_type=jnp.float32)
            cw1_ref[...] = jnp.dot(t1, ax_ref[...],
                                   preferred_element_type=jnp.float32)
        base = 192 + (i - _S1) * _BC1
        for s in range(0, _BC1, 16):
            x = f1_ref[0, s:s + 16].reshape(16 * 96, 96)
            b = jnp.dot(x, cw1_ref[...], preferred_element_type=jnp.float32)
            pooled = jnp.sum(b.reshape(16, 96, _K) * rw1_ref[...][None],
                             axis=1)
            acc_ref[pl.ds(base + s, 16), :] = pooled + pos_embed(base + s, 16)

    @pl.when((i >= _S2) & (i < _S3))
    def _level2():
        @pl.when(i == _S2)
        def _():
            ty = _up_tent(48, 2304, 0, 1, wdiv=48)
            tx = _up_tent_mod(48, 48, 2304)
            w2_ref[...] = (
                jnp.dot(ty, ay_ref[...], preferred_element_type=jnp.float32)
                * jnp.dot(tx, ax_ref[...],
                          preferred_element_type=jnp.float32))
        x = f2_ref[...].reshape(_BC2, 2304)
        pooled = jnp.dot(x, w2_ref[...], preferred_element_type=jnp.float32)
        base = 576 + (i - _S2) * _BC2
        acc_ref[pl.ds(base, _BC2), :] = pooled + pos_embed(base, _BC2)

    @pl.when(i >= _S3)
    def _level3():
        @pl.when(i == _S3)
        def _():
            ty = _up_tent(24, 576, 0, 1, wdiv=24)
            tx = _up_tent_mod(24, 24, 576)
            w3_ref[...] = (
                jnp.dot(ty, ay_ref[...], preferred_element_type=jnp.float32)
                * jnp.dot(tx, ax_ref[...],
                          preferred_element_type=jnp.float32))
        x = f3_ref[...].reshape(_BC3, 576)
        pooled = jnp.dot(x, w3_ref[...], preferred_element_type=jnp.float32)
        base = 1344 + (i - _S3) * _BC3
        acc_ref[pl.ds(base, _BC3), :] = pooled + pos_embed(base, _BC3)

    @pl.when(i == _STEPS - 1)
    def _emit():
        o_ref[...] = jnp.transpose(acc_ref[...], (1, 0))


def kernel(feat0, feat1, feat2, feat3, boxes):
    f2 = feat2.reshape(1, 768, 48 * 48)
    f3 = feat3.reshape(1, 1536, 24 * 24)
    full = pl.pallas_call(
        _body,
        out_shape=jax.ShapeDtypeStruct((_K, 2880), jnp.float32),
        grid=(_STEPS,),
        in_specs=[
            pl.BlockSpec((_K, 4), lambda i: (0, 0)),
            pl.BlockSpec((1, _BC0, 192, 192),
                         lambda i: (0, jnp.minimum(i, _S1 - 1), 0, 0)),
            pl.BlockSpec((1, _BC1, 96, 96),
                         lambda i: (0, jnp.clip(i - _S1, 0, _N1 - 1), 0, 0)),
            pl.BlockSpec((1, _BC2, 2304),
                         lambda i: (0, jnp.clip(i - _S2, 0, _N2 - 1), 0)),
            pl.BlockSpec((1, _BC3, 576),
                         lambda i: (0, jnp.clip(i - _S3, 0, _N3 - 1), 0)),
        ],
        out_specs=pl.BlockSpec((_K, 2880), lambda i: (0, 0)),
        scratch_shapes=[
            pltpu.VMEM((4, _K), jnp.float32),
            pltpu.VMEM((2880, _K), jnp.float32),
            pltpu.VMEM((192, _K), jnp.float32),
            pltpu.VMEM((192, _K), jnp.float32),
            pltpu.VMEM((192, _K), jnp.float32),
            pltpu.VMEM((192, _K), jnp.float32),
            pltpu.VMEM((96, _K), jnp.float32),
            pltpu.VMEM((96, _K), jnp.float32),
            pltpu.VMEM((2304, _K), jnp.float32),
            pltpu.VMEM((576, _K), jnp.float32),
        ],
        compiler_params=pltpu.CompilerParams(
            dimension_semantics=("arbitrary",)),
        name="roi_fused",
    )(boxes, feat0, feat1, f2, f3)
    return full[None]
